# Initial kernel scaffold; baseline (speedup 1.0000x reference)
import jax
import jax.numpy as jnp
from jax.experimental import pallas as pl


def kernel(batch, table):
    raise NotImplementedError("write your pallas kernel here")


# speedup vs baseline: 3.1825x; 3.1825x over previous
import functools

import jax
import jax.numpy as jnp
from jax import lax
from jax.experimental import pallas as pl
from jax.experimental.pallas import tpu as pltpu
from jax.experimental.pallas import tpu_sc as plsc

DIM = 16
MAX_POWER = 99
N = 3
OVERFLOW = (MAX_POWER + 1) ** N
BLK = 128
G = 8
LANES = 16


@functools.cache
def _build_sc_gather(rows: int, vocab: int):
    info = plsc.get_sparse_core_info()
    nw = info.num_cores * info.num_subcores
    nblk = rows // BLK
    assert rows % BLK == 0 and nblk % (nw * G) == 0
    pw = nblk // nw
    chunks = pw // G
    mesh = plsc.VectorSubcoreMesh(core_axis_name="c", subcore_axis_name="s")

    @functools.partial(
        pl.kernel,
        mesh=mesh,
        out_type=jax.ShapeDtypeStruct((nblk, BLK, DIM), jnp.float32),
        scratch_types=[
            pltpu.VMEM((3 * G * BLK,), jnp.int32),
            pltpu.VMEM((G, BLK), jnp.int32),
            pltpu.VMEM((G, BLK, DIM), jnp.float32),
            pltpu.SemaphoreType.DMA,
        ],
        compiler_params=pltpu.CompilerParams(
            needs_layout_passes=False, use_tc_tiling_on_sc=False
        ),
    )
    def sc_gather(batch_hbm, table_hbm, out_hbm, bv, idx_v, rows_v, sem):
        wid = lax.axis_index("s") * info.num_cores + lax.axis_index("c")
        lane = lax.iota(jnp.int32, LANES)

        def chunk_body(c, carry):
            block_base = wid * pw + c * G
            row_base = block_base * BLK
            pltpu.sync_copy(batch_hbm.at[pl.ds(row_base * 3, 3 * G * BLK)], bv)
            for g in range(G):
                for k in range(BLK // LANES):
                    off = (g * BLK + k * LANES) * 3
                    i0 = lane * 3 + off
                    m0 = plsc.load_gather(bv, [i0])
                    m1 = plsc.load_gather(bv, [i0 + 1])
                    m2 = plsc.load_gather(bv, [i0 + 2])
                    m0 = m0 + (m0 == -1).astype(jnp.int32)
                    m1 = m1 + (m1 == -1).astype(jnp.int32)
                    m2 = m2 + (m2 == -1).astype(jnp.int32)
                    enc = m0 + m1 * 100 + m2 * 10000
                    mx = jnp.maximum(jnp.maximum(m0, m1), m2)
                    enc = jnp.where(mx <= MAX_POWER, enc, OVERFLOW)
                    idx_v[g, pl.ds(k * LANES, LANES)] = enc
            copies = [
                pltpu.async_copy(table_hbm.at[idx_v.at[g]], rows_v.at[g], sem)
                for g in range(G)
            ]
            for cp in copies:
                cp.wait()
            pltpu.sync_copy(rows_v, out_hbm.at[pl.ds(block_base, G)])
            return carry

        lax.fori_loop(0, chunks, chunk_body, 0)

    return sc_gather


def kernel(batch, table):
    b, s, w = batch.shape
    rows = (b * s * w) // N
    flat = batch.reshape(rows * N)
    out = _build_sc_gather(rows, table.shape[0])(flat, table)
    return out.reshape(b, s, (w // N) * DIM)


# speedup vs baseline: 3.5502x; 1.1155x over previous
import functools

import jax
import jax.numpy as jnp
from jax import lax
from jax.experimental import pallas as pl
from jax.experimental.pallas import tpu as pltpu
from jax.experimental.pallas import tpu_sc as plsc

DIM = 16
MAX_POWER = 99
N = 3
OVERFLOW = (MAX_POWER + 1) ** N
BLK = 128
G = 16
LANES = 16


@functools.cache
def _build_sc_gather(rows: int, vocab: int):
    info = plsc.get_sparse_core_info()
    nw = info.num_cores * info.num_subcores
    nblk = rows // BLK
    assert rows % BLK == 0 and nblk % (nw * G) == 0
    pw = nblk // nw
    chunks = pw // G
    assert chunks >= 2
    cb = 3 * G * BLK
    mesh = plsc.VectorSubcoreMesh(core_axis_name="c", subcore_axis_name="s")

    @functools.partial(
        pl.kernel,
        mesh=mesh,
        out_type=jax.ShapeDtypeStruct((nblk, BLK, DIM), jnp.float32),
        scratch_types=[
            pltpu.VMEM((2 * cb,), jnp.int32),
            pltpu.VMEM((G, BLK), jnp.int32),
            pltpu.VMEM((2 * G, BLK, DIM), jnp.float32),
            pltpu.SemaphoreType.DMA,
            pltpu.SemaphoreType.DMA,
            pltpu.SemaphoreType.DMA,
        ],
        compiler_params=pltpu.CompilerParams(
            needs_layout_passes=False, use_tc_tiling_on_sc=False
        ),
    )
    def sc_gather(batch_hbm, table_hbm, out_hbm, bv, idx_v, rows_v, bsem, gsem, wsem):
        wid = lax.axis_index("s") * info.num_cores + lax.axis_index("c")
        lane = lax.iota(jnp.int32, LANES)

        def batch_slice(c):
            return batch_hbm.at[pl.ds((wid * pw + c * G) * BLK * 3, cb)]

        pltpu.async_copy(batch_slice(0), bv.at[pl.ds(0, cb)], bsem)

        def chunk_body(c, carry):
            p = lax.rem(c, 2)
            block_base = wid * pw + c * G
            pltpu.make_async_copy(
                batch_slice(0), bv.at[pl.ds(p * cb, cb)], bsem
            ).wait()

            @pl.when(c >= 2)
            def _():
                pltpu.make_async_copy(
                    rows_v.at[pl.ds(0, G)], out_hbm.at[pl.ds(0, G)], wsem
                ).wait()

            base = p * cb
            gathers = []
            for g in range(G):
                for k in range(BLK // LANES):
                    i0 = lane * 3 + (base + (g * BLK + k * LANES) * 3)
                    m0 = plsc.load_gather(bv, [i0])
                    m1 = plsc.load_gather(bv, [i0 + 1])
                    m2 = plsc.load_gather(bv, [i0 + 2])
                    m0 = m0 + (m0 == -1).astype(jnp.int32)
                    m1 = m1 + (m1 == -1).astype(jnp.int32)
                    m2 = m2 + (m2 == -1).astype(jnp.int32)
                    enc = m0 + m1 * 100 + m2 * 10000
                    mx = jnp.maximum(jnp.maximum(m0, m1), m2)
                    enc = jnp.where(mx <= MAX_POWER, enc, OVERFLOW)
                    idx_v[g, pl.ds(k * LANES, LANES)] = enc
                gathers.append(
                    pltpu.async_copy(
                        table_hbm.at[idx_v.at[g]], rows_v.at[p * G + g], gsem
                    )
                )

            @pl.when(c + 1 < chunks)
            def _():
                pltpu.async_copy(
                    batch_slice(c + 1),
                    bv.at[pl.ds(lax.rem(c + 1, 2) * cb, cb)],
                    bsem,
                )

            for cp in gathers:
                cp.wait()
            pltpu.async_copy(
                rows_v.at[pl.ds(p * G, G)],
                out_hbm.at[pl.ds(block_base, G)],
                wsem,
            )
            return carry

        lax.fori_loop(0, chunks, chunk_body, 0)
        for _ in range(2):
            pltpu.make_async_copy(
                rows_v.at[pl.ds(0, G)], out_hbm.at[pl.ds(0, G)], wsem
            ).wait()

    return sc_gather


def kernel(batch, table):
    b, s, w = batch.shape
    rows = (b * s * w) // N
    flat = batch.reshape(rows * N)
    out = _build_sc_gather(rows, table.shape[0])(flat, table)
    return out.reshape(b, s, (w // N) * DIM)
